# baseline (speedup 1.0000x reference)
import functools
import math

import jax
import jax.numpy as jnp
from jax.experimental import pallas as pl
from jax.experimental.pallas import tpu as pltpu

_VOCAB = 1000
_D = 4096
_MAX_SEQ = 8192
_HALF_DIM = 64
_THETA = 10000.0
_G = 8


def _gather_body(idx_ref, *refs):
    in_refs = refs[:_G]
    emb_ref, cos_ref, sin_ref = refs[_G:]
    for k in range(_G):
        emb_ref[pl.ds(k, 1), :] = in_refs[k][0]
    i = pl.program_id(0)
    t = (i * _G).astype(jnp.float32) + jax.lax.broadcasted_iota(
        jnp.int32, (_G, _HALF_DIM), 0).astype(jnp.float32)
    j = jax.lax.broadcasted_iota(
        jnp.int32, (_G, _HALF_DIM), 1).astype(jnp.float32)
    inv_freq = jnp.exp(j * (-2.0 * math.log(_THETA) / 128.0))
    freqs = t * inv_freq
    cos_ref[...] = jnp.cos(freqs)
    sin_ref[...] = jnp.sin(freqs)


def _pos_body(out_ref):
    out_ref[...] = jax.lax.broadcasted_iota(jnp.int32, out_ref.shape, 1)


@jax.jit
def kernel(input_ids, token_embeddings):
    b, s = input_ids.shape
    n = b * s
    flat_ids = input_ids.reshape(n)
    grid = n // _G

    grid_spec = pltpu.PrefetchScalarGridSpec(
        num_scalar_prefetch=1,
        grid=(grid,),
        in_specs=[
            pl.BlockSpec(
                (1, 1, _D),
                functools.partial(
                    lambda k, i, idx_ref: (idx_ref[i * _G + k], 0, 0), k))
            for k in range(_G)
        ],
        out_specs=[
            pl.BlockSpec((_G, _D), lambda i, idx_ref: (i, 0)),
            pl.BlockSpec((_G, _HALF_DIM), lambda i, idx_ref: (i, 0)),
            pl.BlockSpec((_G, _HALF_DIM), lambda i, idx_ref: (i, 0)),
        ],
    )
    emb, cos_f, sin_f = pl.pallas_call(
        _gather_body,
        grid_spec=grid_spec,
        out_shape=[
            jax.ShapeDtypeStruct((n, _D), jnp.float32),
            jax.ShapeDtypeStruct((_MAX_SEQ, _HALF_DIM), jnp.float32),
            jax.ShapeDtypeStruct((_MAX_SEQ, _HALF_DIM), jnp.float32),
        ],
    )(flat_ids, *([token_embeddings.reshape(_VOCAB, 1, _D)] * _G))

    position_ids = pl.pallas_call(
        _pos_body,
        out_shape=jax.ShapeDtypeStruct((b, s), jnp.int32),
    )()

    return emb.reshape(b, s, _D), position_ids, cos_f, sin_f


# speedup vs baseline: 5.0741x; 5.0741x over previous
import math

import jax
import jax.numpy as jnp
from jax import lax
from jax.experimental import pallas as pl
from jax.experimental.pallas import tpu as pltpu
from jax.experimental.pallas import tpu_sc as plsc

_VOCAB = 1000
_D = 4096
_MAX_SEQ = 8192
_HALF_DIM = 64
_THETA = 10000.0

_NC = 2
_NS = 16
_NW = _NC * _NS
_N_TOK = 8192
_ROWS_PER_W = _N_TOK // _NW
_C = 8
_NCHUNK = _ROWS_PER_W // _C


def _sc_gather_body(table_hbm, idx_hbm, out_hbm, idx_v, buf0, buf1,
                    sem0, sem1):
    wid = lax.axis_index("s") * _NC + lax.axis_index("c")
    base = wid * _ROWS_PER_W
    pltpu.sync_copy(idx_hbm.at[pl.ds(base, _ROWS_PER_W)], idx_v)
    bufs = (buf0, buf1)
    sems = (sem0, sem1)

    def start(c, b):
        pltpu.async_copy(
            table_hbm.at[idx_v.at[pl.ds(c * _C, _C)]], bufs[b], sems[b])

    start(0, 0)
    start(1, 1)

    def step(t, carry):
        for b in range(2):
            c = 2 * t + b
            pltpu.make_async_copy(
                table_hbm.at[idx_v.at[pl.ds(c * _C, _C)]],
                bufs[b], sems[b]).wait()
            pltpu.sync_copy(bufs[b], out_hbm.at[pl.ds(base + c * _C, _C)])

            @pl.when(t < _NCHUNK // 2 - 1)
            def _():
                start(c + 2, b)
        return carry

    lax.fori_loop(0, _NCHUNK // 2, step, 0)


_sc_gather = pl.kernel(
    _sc_gather_body,
    out_type=jax.ShapeDtypeStruct((_N_TOK, _D), jnp.float32),
    mesh=plsc.VectorSubcoreMesh(core_axis_name="c", subcore_axis_name="s"),
    scratch_types=[
        pltpu.VMEM((_ROWS_PER_W,), jnp.int32),
        pltpu.VMEM((_C, _D), jnp.float32),
        pltpu.VMEM((_C, _D), jnp.float32),
        pltpu.SemaphoreType.DMA,
        pltpu.SemaphoreType.DMA,
    ],
)


def _rope_body(cos_ref, sin_ref, pos_ref):
    i = pl.program_id(0)
    rows = pl.num_programs(0)
    blk = cos_ref.shape[0]
    lane = jax.lax.broadcasted_iota(jnp.int32, (blk, 128), 1)
    row = i * blk + jax.lax.broadcasted_iota(jnp.int32, (blk, 128), 0)
    t = (2 * row + (lane >= _HALF_DIM).astype(jnp.int32)).astype(jnp.float32)
    j = (lane & (_HALF_DIM - 1)).astype(jnp.float32)
    inv_freq = jnp.exp(j * (-2.0 * math.log(_THETA) / 128.0))
    freqs = t * inv_freq
    cos_ref[...] = jnp.cos(freqs)
    sin_ref[...] = jnp.sin(freqs)

    @pl.when(i == 0)
    def _():
        pos_ref[...] = jax.lax.broadcasted_iota(
            jnp.int32, pos_ref.shape, 1)


_ROPE_BLK = 512
_ROPE_GRID = (_MAX_SEQ // 2) // _ROPE_BLK


@jax.jit
def kernel(input_ids, token_embeddings):
    b, s = input_ids.shape
    flat_ids = input_ids.reshape(b * s)

    emb = _sc_gather(token_embeddings, flat_ids)

    cos_d, sin_d, position_ids = pl.pallas_call(
        _rope_body,
        grid=(_ROPE_GRID,),
        out_specs=[
            pl.BlockSpec((_ROPE_BLK, 128), lambda i: (i, 0)),
            pl.BlockSpec((_ROPE_BLK, 128), lambda i: (i, 0)),
            pl.BlockSpec((b, s), lambda i: (0, 0)),
        ],
        out_shape=[
            jax.ShapeDtypeStruct((_MAX_SEQ // 2, 128), jnp.float32),
            jax.ShapeDtypeStruct((_MAX_SEQ // 2, 128), jnp.float32),
            jax.ShapeDtypeStruct((b, s), jnp.int32),
        ],
    )()

    return (emb.reshape(b, s, _D), position_ids,
            cos_d.reshape(_MAX_SEQ, _HALF_DIM),
            sin_d.reshape(_MAX_SEQ, _HALF_DIM))
